# baseline (speedup 1.0000x reference)
import functools

import jax
import jax.numpy as jnp
from jax import lax
from jax.experimental import pallas as pl
from jax.experimental.pallas import tpu as pltpu
from jax.experimental.pallas import tpu_sc as plsc

N_VERTS = 5023
N_CROP = 1787
BATCH = 4096
ROW = 3 * N_VERTS
OUT_ROW = 3 * N_CROP
LANES = 16
OUT_PAD = ((OUT_ROW + LANES - 1) // LANES) * LANES
N_CHUNK = OUT_PAD // LANES
NUM_CORES = 2
NUM_SUBCORES = 16
NW = NUM_CORES * NUM_SUBCORES
B_PER_W = BATCH // NW


def _make_gather():
    mesh = plsc.VectorSubcoreMesh(core_axis_name="c", subcore_axis_name="s")

    @functools.partial(
        pl.kernel,
        mesh=mesh,
        compiler_params=pltpu.CompilerParams(
            needs_layout_passes=False, use_tc_tiling_on_sc=False),
        out_type=jax.ShapeDtypeStruct((BATCH, OUT_ROW), jnp.float32),
        scratch_types=[
            pltpu.VMEM((OUT_PAD,), jnp.int32),
            pltpu.VMEM((ROW,), jnp.float32),
            pltpu.VMEM((OUT_PAD,), jnp.float32),
        ],
    )
    def gather_kernel(x_hbm, eidx_hbm, out_hbm, eidx_v, xb_v, outb_v):
        wid = lax.axis_index("s") * NUM_CORES + lax.axis_index("c")
        pltpu.sync_copy(eidx_hbm, eidx_v)

        def batch_body(bi, _):
            b = wid * B_PER_W + bi
            pltpu.sync_copy(x_hbm.at[b], xb_v)

            def chunk_body(i, _):
                idxc = eidx_v[pl.ds(i * LANES, LANES)]
                outb_v[pl.ds(i * LANES, LANES)] = plsc.load_gather(
                    xb_v, [idxc])
                return 0

            lax.fori_loop(0, N_CHUNK, chunk_body, 0, unroll=4)
            pltpu.sync_copy(outb_v.at[pl.ds(0, OUT_ROW)], out_hbm.at[b])
            return 0

        lax.fori_loop(0, B_PER_W, batch_body, 0)

    return gather_kernel


_gather = _make_gather()


def kernel(x, vidx):
    vidx32 = vidx.astype(jnp.int32)
    eidx = (vidx32[:, None] * 3 + jnp.arange(3, dtype=jnp.int32)).reshape(-1)
    eidx = jnp.concatenate(
        [eidx, jnp.zeros((OUT_PAD - OUT_ROW,), jnp.int32)])
    x2 = x.reshape(BATCH, ROW)
    out = _gather(x2, eidx)
    return out.reshape(BATCH, N_CROP, 3)


# speedup vs baseline: 16.6019x; 16.6019x over previous
import functools

import jax
import jax.numpy as jnp
from jax import lax
from jax.experimental import pallas as pl
from jax.experimental.pallas import tpu as pltpu
from jax.experimental.pallas import tpu_sc as plsc

N_VERTS = 5023
N_CROP = 1787
BATCH = 4096
NUM_CORES = 2
NUM_SUBCORES = 16
NW = NUM_CORES * NUM_SUBCORES
K = 8
N_BANDS = (N_CROP + K - 1) // K
BANDS_PER_W = N_BANDS // NW
IDX_PAD = N_BANDS * K
TAIL_BAND = N_BANDS - 1
TAIL_ROWS = N_CROP - TAIL_BAND * K


def _make_gather():
    mesh = plsc.VectorSubcoreMesh(core_axis_name="c", subcore_axis_name="s")

    @functools.partial(
        pl.kernel,
        mesh=mesh,
        out_type=jax.ShapeDtypeStruct((3, N_CROP, BATCH), jnp.float32),
        scratch_types=[
            pltpu.VMEM((IDX_PAD,), jnp.int32),
            pltpu.VMEM((K, BATCH), jnp.float32),
            pltpu.SemaphoreType.DMA,
        ],
    )
    def gather_kernel(xt, vidx_hbm, out_hbm, idx_v, buf, sem):
        wid = lax.axis_index("s") * NUM_CORES + lax.axis_index("c")
        pltpu.sync_copy(vidx_hbm, idx_v)
        for c in range(3):
            xc = xt.at[c]
            for i in range(BANDS_PER_W):
                g = wid * BANDS_PER_W + i
                pltpu.async_copy(
                    xc.at[idx_v.at[pl.ds(g * K, K)]], buf, sem).wait()

                @pl.when(g != TAIL_BAND)
                def _():
                    pltpu.sync_copy(
                        buf, out_hbm.at[c, pl.ds(g * K, K), :])

                @pl.when(g == TAIL_BAND)
                def _():
                    pltpu.sync_copy(
                        buf.at[pl.ds(0, TAIL_ROWS)],
                        out_hbm.at[c, pl.ds(TAIL_BAND * K, TAIL_ROWS), :])

    return gather_kernel


_gather = _make_gather()


def kernel(x, vidx):
    vidx32 = vidx.astype(jnp.int32)
    vpad = jnp.concatenate(
        [vidx32, jnp.zeros((IDX_PAD - N_CROP,), jnp.int32)])
    xt = x.transpose(2, 1, 0)
    out = _gather(xt, vpad)
    return out.transpose(2, 1, 0)


# speedup vs baseline: 18.5366x; 1.1165x over previous
import functools

import jax
import jax.numpy as jnp
from jax import lax
from jax.experimental import pallas as pl
from jax.experimental.pallas import tpu as pltpu
from jax.experimental.pallas import tpu_sc as plsc

N_VERTS = 5023
N_CROP = 1787
BATCH = 4096
NUM_CORES = 2
NUM_SUBCORES = 16
NW = NUM_CORES * NUM_SUBCORES
K = 8
N_BANDS = (N_CROP + K - 1) // K
BANDS_PER_W = N_BANDS // NW
IDX_PAD = N_BANDS * K
TAIL_BAND = N_BANDS - 1
TAIL_ROWS = N_CROP - TAIL_BAND * K


def _make_gather():
    mesh = plsc.VectorSubcoreMesh(core_axis_name="c", subcore_axis_name="s")

    @functools.partial(
        pl.kernel,
        mesh=mesh,
        out_type=jax.ShapeDtypeStruct((3, N_CROP, BATCH), jnp.float32),
        scratch_types=[
            pltpu.VMEM((IDX_PAD,), jnp.int32),
            pltpu.VMEM((3, K, BATCH), jnp.float32),
            pltpu.SemaphoreType.DMA,
            pltpu.SemaphoreType.DMA,
            pltpu.SemaphoreType.DMA,
            pltpu.SemaphoreType.DMA,
            pltpu.SemaphoreType.DMA,
            pltpu.SemaphoreType.DMA,
        ],
    )
    def gather_kernel(xt, vidx_hbm, out_hbm, idx_v, bufs, s0, s1, s2, t0,
                      t1, t2, ):
        wid = lax.axis_index("s") * NUM_CORES + lax.axis_index("c")
        sem_in = (s0, s1, s2)
        sem_out = (t0, t1, t2)
        pltpu.sync_copy(vidx_hbm, idx_v)

        tasks = [(c, i) for c in range(3) for i in range(BANDS_PER_W)]
        n = len(tasks)
        depth = 3

        def in_copy(t):
            c, i = tasks[t]
            g = wid * BANDS_PER_W + i
            return pltpu.make_async_copy(
                xt.at[c].at[idx_v.at[pl.ds(g * K, K)]],
                bufs.at[t % depth], sem_in[t % depth])

        def start_out(t):
            c, i = tasks[t]
            g = wid * BANDS_PER_W + i
            full = pltpu.make_async_copy(
                bufs.at[t % depth], out_hbm.at[c, pl.ds(g * K, K), :],
                sem_out[t % depth])
            tail = pltpu.make_async_copy(
                bufs.at[t % depth].at[pl.ds(0, TAIL_ROWS)],
                out_hbm.at[c, pl.ds(TAIL_BAND * K, TAIL_ROWS), :],
                sem_out[t % depth])

            @pl.when(g != TAIL_BAND)
            def _():
                full.start()

            @pl.when(g == TAIL_BAND)
            def _():
                tail.start()
            return full, tail

        def wait_out(t, handles):
            c, i = tasks[t]
            g = wid * BANDS_PER_W + i
            full, tail = handles

            @pl.when(g != TAIL_BAND)
            def _():
                full.wait()

            @pl.when(g == TAIL_BAND)
            def _():
                tail.wait()

        for t in range(depth):
            in_copy(t).start()
        pending = {}
        for t in range(n):
            in_copy(t).wait()
            pending[t] = start_out(t)
            if t + depth < n:
                wait_out(t, pending.pop(t))
                in_copy(t + depth).start()
        for t in sorted(pending):
            wait_out(t, pending[t])

    return gather_kernel


_gather = _make_gather()


def kernel(x, vidx):
    vidx32 = vidx.astype(jnp.int32)
    vpad = jnp.concatenate(
        [vidx32, jnp.zeros((IDX_PAD - N_CROP,), jnp.int32)])
    xt = x.transpose(2, 1, 0)
    out = _gather(xt, vpad)
    return out.transpose(2, 1, 0)
